# Initial kernel scaffold; baseline (speedup 1.0000x reference)
import jax
import jax.numpy as jnp
from jax.experimental import pallas as pl


def kernel(UVs, VUs, params):
    raise NotImplementedError("write your pallas kernel here")


# speedup vs baseline: 8.4387x; 8.4387x over previous
import functools

import jax
import jax.numpy as jnp
from jax import lax
from jax.experimental import pallas as pl
from jax.experimental.pallas import tpu as pltpu
from jax.experimental.pallas import tpu_sc as plsc

NU = 10000
FD = 128
FH = 64
NP = 10112
ALPHA = 0.1

NC, NS = 2, 16
NW = NC * NS
CH = 128
CPT = 160
CPW = 80
NCK = NS * CPT
EP = NCK * CH
RPT = NP // NS
KG = 2
NG = CPT // KG

_MESH = plsc.VectorSubcoreMesh(core_axis_name="c", subcore_axis_name="s",
                               num_cores=NC, num_subcores=NS)


def _spmm_body(x_hbm, idx_hbm, zeros_hbm, out_hbm,
               ib0, ib1, rows, acc, xs, gsem0, gsem1, ssem0, ssem1,
               isem0, isem1):
    cid = lax.axis_index("c")
    tid = lax.axis_index("s")
    pltpu.sync_copy(zeros_hbm.at[pl.ds(tid * RPT, RPT)],
                    acc.at[pl.ds(tid * RPT, RPT)])
    pltpu.sync_copy(x_hbm.at[cid].at[pl.ds(tid * RPT, RPT)],
                    xs.at[pl.ds(tid * RPT, RPT)])
    plsc.subcore_barrier()
    xc = xs
    ibufs = [ib0, ib1]
    gsems = [gsem0, gsem1]
    ssems = [ssem0, ssem1]
    isems = [isem0, isem1]

    def idx_slice(g):
        return idx_hbm.at[pl.ds(tid * CPT + g * KG, KG)]

    def fire_i(g, bank):
        pltpu.async_copy(idx_slice(g), ibufs[bank], isems[bank])

    def wait_i(g, bank):
        pltpu.make_async_copy(idx_slice(g), ibufs[bank], isems[bank]).wait()

    def fire_g(g, bank):
        for b in range(KG):
            pltpu.async_copy(xc.at[ibufs[bank].at[b, 0]],
                             rows.at[bank * KG + b], gsems[bank])

    def wait_g(g, bank):
        for b in range(KG):
            pltpu.make_async_copy(xc.at[ibufs[bank].at[b, 0]],
                                  rows.at[bank * KG + b], gsems[bank]).wait()

    def fire_s(g, bank):
        for b in range(KG):
            pltpu.async_copy(rows.at[bank * KG + b],
                             acc.at[ibufs[bank].at[b, 1]], ssems[bank],
                             add=True)

    def wait_s(g, bank):
        for b in range(KG):
            pltpu.make_async_copy(rows.at[bank * KG + b],
                                  acc.at[ibufs[bank].at[b, 1]],
                                  ssems[bank]).wait()

    pltpu.sync_copy(idx_slice(0), ib0)
    fire_g(0, 0)

    def pair(i, carry):
        g0 = 2 * i
        g1 = g0 + 1

        @pl.when(i > 0)
        def _():
            wait_s(g0 - 1, 1)

        fire_i(g1, 1)
        wait_g(g0, 0)
        fire_s(g0, 0)
        wait_i(g1, 1)
        fire_g(g1, 1)
        wait_s(g0, 0)

        @pl.when(i < NG // 2 - 1)
        def _():
            fire_i(g0 + 2, 0)

        wait_g(g1, 1)
        fire_s(g1, 1)

        @pl.when(i < NG // 2 - 1)
        def _():
            wait_i(g0 + 2, 0)
            fire_g(g0 + 2, 0)
        return carry

    lax.fori_loop(0, NG // 2, pair, 0)
    wait_s(NG - 1, 1)
    plsc.subcore_barrier()
    pltpu.sync_copy(acc.at[pl.ds(tid * RPT, RPT)],
                    out_hbm.at[cid].at[pl.ds(tid * RPT, RPT)])


_SPMM = pl.kernel(
    _spmm_body,
    out_type=jax.ShapeDtypeStruct((NC, NP, FH), jnp.float32),
    mesh=_MESH,
    scratch_types=[
        pltpu.VMEM((KG, 2, CH), jnp.int32),
        pltpu.VMEM((KG, 2, CH), jnp.int32),
        pltpu.VMEM((2 * KG, CH, FH), jnp.float32),
        pltpu.VMEM_SHARED((NP, FH), jnp.float32),
        pltpu.VMEM_SHARED((NP, FH), jnp.float32),
        pltpu.SemaphoreType.DMA,
        pltpu.SemaphoreType.DMA,
        pltpu.SemaphoreType.DMA,
        pltpu.SemaphoreType.DMA,
        pltpu.SemaphoreType.DMA,
        pltpu.SemaphoreType.DMA,
    ],
    compiler_params=pltpu.CompilerParams(use_tc_tiling_on_sc=False),
)


def _deg_body(idx_hbm, ones_hbm, zeros_hbm, out_hbm, didx, ones_v, acc, dsem):
    cid = lax.axis_index("c")
    tid = lax.axis_index("s")
    wid = cid * NS + tid
    pltpu.sync_copy(idx_hbm.at[pl.ds(wid * CPW, CPW)], didx)
    pltpu.sync_copy(ones_hbm, ones_v)
    pltpu.sync_copy(zeros_hbm.at[pl.ds(tid * RPT, RPT)],
                    acc.at[pl.ds(tid * RPT, RPT)])
    plsc.subcore_barrier()

    def group(g, carry):
        base = g * 8
        descs = [pltpu.async_copy(ones_v, acc.at[didx.at[base + b, 1]], dsem,
                                  add=True) for b in range(8)]
        for b in range(8):
            descs[b].wait()
        return carry

    lax.fori_loop(0, CPW // 8, group, 0)
    plsc.subcore_barrier()
    pltpu.sync_copy(acc.at[pl.ds(tid * RPT, RPT)],
                    out_hbm.at[cid].at[pl.ds(tid * RPT, RPT)])


_DEG = pl.kernel(
    _deg_body,
    out_type=jax.ShapeDtypeStruct((NC, NP, 16), jnp.float32),
    mesh=_MESH,
    scratch_types=[
        pltpu.VMEM((CPW, 2, CH), jnp.int32),
        pltpu.VMEM((CH, 16), jnp.float32),
        pltpu.VMEM_SHARED((NP, 16), jnp.float32),
        pltpu.SemaphoreType.DMA,
    ],
    compiler_params=pltpu.CompilerParams(use_tc_tiling_on_sc=False),
)


def _scale_mm_body(act, part, d0, d1, w, b, o):
    deg = d0[:, 0:1] + d1[:, 0:1]
    inv = 1.0 / jnp.maximum(deg, 1.0)
    s = jnp.concatenate([part[0], part[1]], axis=1) * inv
    y = jnp.dot(s, w[...], preferred_element_type=jnp.float32) + b[...]
    if act == "lrelu":
        y = jnp.where(y >= 0, y, ALPHA * y)
    elif act == "relu":
        y = jnp.maximum(y, 0.0)
    o[0] = y[:, :FH]
    o[1] = y[:, FH:]


@functools.lru_cache(maxsize=None)
def _scale_mm(act):
    return pl.pallas_call(
        functools.partial(_scale_mm_body, act),
        out_shape=jax.ShapeDtypeStruct((NC, NP, FH), jnp.float32),
    )


def _comb_body(act, has_add, full_out, *refs):
    if has_add:
        x1, x2, w1, w2, b, ad, o = refs
    else:
        x1, x2, w1, w2, b, o = refs
        ad = None
    c1 = jnp.concatenate([x1[0], x1[1]], axis=1)
    c2 = jnp.concatenate([x2[0], x2[1]], axis=1)
    y = (jnp.dot(c1, w1[...], preferred_element_type=jnp.float32)
         + jnp.dot(c2, w2[...], preferred_element_type=jnp.float32)
         + b[...])
    if ad is not None:
        y = y + jnp.concatenate([ad[0], ad[1]], axis=1)
    if act == "relu":
        y = jnp.maximum(y, 0.0)
    if full_out:
        o[...] = y
    else:
        o[0] = y[:, :FH]
        o[1] = y[:, FH:]


@functools.lru_cache(maxsize=None)
def _comb(act, has_add, full_out):
    shape = (NP, FD) if full_out else (NC, NP, FH)
    return pl.pallas_call(
        functools.partial(_comb_body, act, has_add, full_out),
        out_shape=jax.ShapeDtypeStruct(shape, jnp.float32),
    )


def _prep_edges(src, dst):
    e = src.shape[0]
    pad = EP - e
    src = jnp.concatenate([src, jnp.zeros((pad,), jnp.int32)])
    dst = jnp.concatenate([dst, jnp.full((pad,), NU, jnp.int32)])
    return jnp.stack([src.reshape(NCK, CH), dst.reshape(NCK, CH)], axis=1)


def _split_cols(x):
    xp = jnp.pad(x, ((0, NP - x.shape[0]), (0, 0)))
    return xp.reshape(NP, NC, FH).transpose(1, 0, 2)


def kernel(UVs, VUs, params):
    del VUs
    p = params
    edges = {}
    for k in range(2):
        r, c = UVs[k, 0], UVs[k, 1]
        edges[("uv", k)] = _prep_edges(c, r)
        edges[("vu", k)] = _prep_edges(r, c)

    zeros = jnp.zeros((NP, FH), jnp.float32)
    zeros16 = jnp.zeros((NP, 16), jnp.float32)
    ones16 = jnp.ones((CH, 16), jnp.float32)

    deg = {}
    for key, idx2 in edges.items():
        dp = _DEG(idx2, ones16, zeros16)
        deg[key] = (dp[0], dp[1])

    def gcn(key, x, W, bvec):
        part = _SPMM(x, edges[key], zeros)
        d0, d1 = deg[key]
        return _scale_mm("lrelu")(part, d0, d1, W, bvec.reshape(1, FD))

    def comb(x1, x2, W, bvec, act="none", addend=None, full_out=False):
        args = [x1, x2, W[:FD], W[FD:], bvec.reshape(1, FD)]
        if addend is not None:
            args.append(addend)
        return _comb(act, addend is not None, full_out)(*args)

    spec_u, spec_i = [], []
    for k in range(2):
        l0, l1 = p["spec"][k]["l0"], p["spec"][k]["l1"]
        ue = _split_cols(p["user_emb"][k])
        ie = _split_cols(p["item_emb"][k])
        u_ho = gcn(("vu", k), ue, l0["gc1_W"], l0["gc1_b"])
        i_ho = gcn(("uv", k), ie, l0["gc2_W"], l0["gc2_b"])
        u2 = gcn(("uv", k), u_ho, l0["gc3_W"], l0["gc3_b"])
        i2 = gcn(("vu", k), i_ho, l0["gc4_W"], l0["gc4_b"])
        u = comb(u2, ue, l0["uu_W"], l0["uu_b"], "relu")
        v = comb(i2, ie, l0["iu_W"], l0["iu_b"], "relu")
        u_ho2 = gcn(("vu", k), u, l1["gc1_W"], l1["gc1_b"])
        i_ho2 = gcn(("uv", k), v, l1["gc2_W"], l1["gc2_b"])
        um = gcn(("uv", k), u_ho2, l1["gc3m_W"], l1["gc3m_b"])
        im = gcn(("vu", k), i_ho2, l1["gc4m_W"], l1["gc4m_b"])
        spec_u.append((um, u, l1["uum_W"], l1["uum_b"]))
        spec_i.append(comb(im, v, l1["ium_W"], l1["ium_b"], full_out=True))

    c0, c1 = p["cross"]["l0"], p["cross"]["l1"]
    cu = []
    for k in range(2):
        ues = _split_cols(p["user_emb_share"][k])
        h = gcn(("vu", k), ues, c0["gc1_W"], c0["gc1_b"])
        h2 = gcn(("uv", k), h, c0["gc3_W"], c0["gc3_b"])
        cu.append(comb(h2, ues, c0["uu_W"], c0["uu_b"], "relu"))
    u0 = gcn(("vu", 0), cu[0], c1["gc1_W"], c1["gc1_b"])
    m0 = gcn(("uv", 0), u0, c1["gc3m_W"], c1["gc3m_b"])
    u1 = gcn(("vu", 1), cu[1], c1["gc2_W"], c1["gc2_b"])
    m1 = gcn(("uv", 1), u1, c1["gc4m_W"], c1["gc4m_b"])
    mean = comb(m0, m1, c1["uum_W"], c1["uum_b"])

    outs = [comb(*spec_u[k], addend=mean, full_out=True)[:NU]
            for k in range(2)]
    outs += [spec_i[k][:NU] for k in range(2)]
    return jnp.stack(outs)
